# Initial kernel scaffold; baseline (speedup 1.0000x reference)
import jax
import jax.numpy as jnp
from jax.experimental import pallas as pl


def kernel(indices, values, emb_table, bias):
    raise NotImplementedError("write your pallas kernel here")


# speedup vs baseline: 1.1380x; 1.1380x over previous
import functools

import jax
import jax.numpy as jnp
from jax import lax
from jax.experimental import pallas as pl
from jax.experimental.pallas import tpu as pltpu
from jax.experimental.pallas import tpu_sc as plsc

B = 16384
L = 100
NC = 2
NS = 16
NW = NC * NS
RW = B // NW
C = 128
NCHUNK = RW // C
G = C // 16
CL = C * L

_mesh = plsc.VectorSubcoreMesh(core_axis_name="c", subcore_axis_name="s")


@functools.partial(
    pl.kernel,
    mesh=_mesh,
    out_type=jax.ShapeDtypeStruct((B,), jnp.float32),
    scratch_types=[
        pltpu.VMEM((CL,), jnp.int32),
        pltpu.VMEM((CL,), jnp.float32),
        pltpu.VMEM((CL,), jnp.float32),
        pltpu.VMEM((C,), jnp.float32),
        pltpu.SemaphoreType.DMA,
    ],
    compiler_params=pltpu.CompilerParams(needs_layout_passes=False),
)
def _wide_sc(idx_hbm, val_hbm, tab_hbm, out_hbm, idx_v, g_v, v_v, o_v, sem):
    wid = lax.axis_index("s") * NC + lax.axis_index("c")
    lane = lax.iota(jnp.int32, 16)

    def chunk_body(k, carry):
        base = wid * RW + k * C
        pltpu.sync_copy(idx_hbm.at[pl.ds(base * L, CL)], idx_v)
        pltpu.sync_copy(val_hbm.at[pl.ds(base * L, CL)], v_v)
        pltpu.async_copy(tab_hbm.at[idx_v], g_v, sem).wait()

        def grp_body(g, carry2):
            row_l = (g * 16 + lane) * L
            acc = jnp.zeros((16,), jnp.float32)
            for slot in range(L):
                fidx = row_l + slot
                gv = plsc.load_gather(g_v, [fidx])
                vv = plsc.load_gather(v_v, [fidx])
                acc = acc + gv * vv
            o_v[pl.ds(g * 16, 16)] = acc
            return carry2

        lax.fori_loop(0, G, grp_body, 0)
        pltpu.sync_copy(o_v, out_hbm.at[pl.ds(base, C)])
        return carry

    lax.fori_loop(0, NCHUNK, chunk_body, 0)


def kernel(indices, values, emb_table, bias):
    idx = indices.astype(jnp.int32).reshape(-1)
    val = values.reshape(-1)
    tab = emb_table.reshape(-1)
    out = _wide_sc(idx, val, tab)
    return out + bias[0]


# speedup vs baseline: 1.1977x; 1.0524x over previous
import functools

import jax
import jax.numpy as jnp
from jax import lax
from jax.experimental import pallas as pl
from jax.experimental.pallas import tpu as pltpu
from jax.experimental.pallas import tpu_sc as plsc

B = 16384
L = 100
NC = 2
NS = 16
NW = NC * NS
RW = B // NW
C = 128
NCHUNK = RW // C
G = C // 16
CL = C * L

_mesh = plsc.VectorSubcoreMesh(core_axis_name="c", subcore_axis_name="s")


@functools.partial(
    pl.kernel,
    mesh=_mesh,
    out_type=jax.ShapeDtypeStruct((B,), jnp.float32),
    scratch_types=[
        pltpu.VMEM((CL,), jnp.int32),
        pltpu.VMEM((CL,), jnp.int32),
        pltpu.VMEM((CL,), jnp.float32),
        pltpu.VMEM((CL,), jnp.float32),
        pltpu.VMEM((C, L), jnp.float32),
        pltpu.VMEM((C, L), jnp.float32),
        pltpu.VMEM((RW,), jnp.float32),
        pltpu.SemaphoreType.DMA,
        pltpu.SemaphoreType.DMA,
        pltpu.SemaphoreType.DMA,
        pltpu.SemaphoreType.DMA,
        pltpu.SemaphoreType.DMA,
        pltpu.SemaphoreType.DMA,
    ],
    compiler_params=pltpu.CompilerParams(needs_layout_passes=False),
)
def _wide_sc(idx_hbm, val_hbm, tab_hbm, out_hbm,
             idx0, idx1, g0, g1, v0, v1, o_v,
             si0, si1, sg0, sg1, sv0, sv1):
    wid = lax.axis_index("s") * NC + lax.axis_index("c")
    lane = lax.iota(jnp.int32, 16)
    idx_b = (idx0, idx1)
    g_b = (g0, g1)
    v_b = (v0, v1)
    si = (si0, si1)
    sg = (sg0, sg1)
    sv = (sv0, sv1)

    def start_idx(k):
        s = k % 2
        return pltpu.async_copy(
            idx_hbm.at[pl.ds((wid * RW + k * C) * L, CL)], idx_b[s], si[s])

    def start_val(k):
        s = k % 2
        return pltpu.async_copy(
            val_hbm.at[pl.ds(wid * RW + k * C, C), :], v_b[s], sv[s])

    def start_gather(k):
        s = k % 2
        return pltpu.async_copy(tab_hbm.at[idx_b[s]], g_b[s], sg[s])

    cp_i0 = start_idx(0)
    cp_v = start_val(0)
    cp_i1 = start_idx(1)
    cp_i0.wait()
    cp_g = start_gather(0)
    cp_i = cp_i1

    for k in range(NCHUNK):
        s = k % 2
        cp_g.wait()
        if k + 1 < NCHUNK:
            cp_i.wait()
            cp_g = start_gather(k + 1)
        if k + 2 < NCHUNK:
            cp_i = start_idx(k + 2)
        cp_v.wait()
        if k + 1 < NCHUNK:
            cp_v = start_val(k + 1)

        g_v, v_v = g_b[s], v_b[s]

        def grp_body(g, carry2, g_v=g_v, v_v=v_v, k=k):
            row = g * 16 + lane
            row_l = row * L
            acc = jnp.zeros((16,), jnp.float32)
            for slot in range(L):
                col = jnp.full((16,), slot, jnp.int32)
                gv = plsc.load_gather(g_v, [row_l + slot])
                vv = plsc.load_gather(v_v, [row, col])
                acc = acc + gv * vv
            o_v[pl.ds(k * C + g * 16, 16)] = acc
            return carry2

        lax.fori_loop(0, G, grp_body, 0)

    pltpu.sync_copy(o_v, out_hbm.at[pl.ds(wid * RW, RW)])


def kernel(indices, values, emb_table, bias):
    idx = indices.astype(jnp.int32).reshape(-1)
    tab = emb_table.reshape(-1)
    out = _wide_sc(idx, values, tab)
    return out + bias[0]


# speedup vs baseline: 1.2301x; 1.0271x over previous
import functools

import jax
import jax.numpy as jnp
from jax import lax
from jax.experimental import pallas as pl
from jax.experimental.pallas import tpu as pltpu
from jax.experimental.pallas import tpu_sc as plsc

B = 16384
L = 100
NC = 2
NS = 16
NW = NC * NS
RW = B // NW
C = 128
NCHUNK = RW // C
G = C // 16
CL = C * L
VSTAGE = 1000000
CB = (L + 15) // 16

_mesh = plsc.VectorSubcoreMesh(core_axis_name="c", subcore_axis_name="s")


@functools.partial(
    pl.kernel,
    mesh=_mesh,
    out_type=jax.ShapeDtypeStruct((B,), jnp.float32),
    scratch_types=[
        pltpu.VMEM((C, L), jnp.int32),
        pltpu.VMEM((C, L), jnp.int32),
        pltpu.VMEM((CL,), jnp.int32),
        pltpu.VMEM((CL,), jnp.int32),
        pltpu.VMEM((CL,), jnp.float32),
        pltpu.VMEM((CL,), jnp.float32),
        pltpu.VMEM((C, L), jnp.float32),
        pltpu.VMEM((C, L), jnp.float32),
        pltpu.VMEM((RW,), jnp.float32),
        pltpu.VMEM((16,), jnp.int32),
        pltpu.VMEM((16,), jnp.float32),
        pltpu.SemaphoreType.DMA,
        pltpu.SemaphoreType.DMA,
        pltpu.SemaphoreType.DMA,
        pltpu.SemaphoreType.DMA,
        pltpu.SemaphoreType.DMA,
        pltpu.SemaphoreType.DMA,
        pltpu.SemaphoreType.DMA,
    ],
    compiler_params=pltpu.CompilerParams(needs_layout_passes=False),
)
def _wide_sc(idx_hbm, val_hbm, tab_hbm, bias_hbm, out_hbm,
             i2_0, i2_1, idx0, idx1, g0, g1, v0, v1, o_v, zidx, bvec,
             si0, si1, sg0, sg1, sv0, sv1, sb):
    cid = lax.axis_index("c")
    sid = lax.axis_index("s")
    wid = sid * NC + cid
    lane = lax.iota(jnp.int32, 16)
    i2_b = (i2_0, i2_1)
    idx_b = (idx0, idx1)
    g_b = (g0, g1)
    v_b = (v0, v1)
    si = (si0, si1)
    sg = (sg0, sg1)
    sv = (sv0, sv1)

    def start_i2(k):
        s = k % 2
        return pltpu.async_copy(
            idx_hbm.at[pl.ds(wid * RW + k * C, C), :], i2_b[s], si[s])

    def start_val(k):
        s = k % 2
        return pltpu.async_copy(
            val_hbm.at[pl.ds(wid * RW + k * C, C), :], v_b[s], sv[s])

    def start_gather(k):
        s = k % 2
        return pltpu.async_copy(tab_hbm.at[idx_b[s]], g_b[s], sg[s])

    def flatten(k):
        s = k % 2
        i2, i1 = i2_b[s], idx_b[s]

        def row_body(r, carry):
            rsplat = jnp.full((16,), r, jnp.int32)
            for cb in range(CB):
                col = cb * 16 + lane
                m = col < L
                x = plsc.load_gather(i2, [rsplat, col], mask=m)
                plsc.store_scatter(i1, [r * L + col], x, mask=m)
            return carry

        lax.fori_loop(0, C, row_body, 0)

    cp_i2_0 = start_i2(0)
    cp_v = start_val(0)
    cp_i2 = start_i2(1)

    zidx[...] = lane * 0
    pltpu.async_copy(bias_hbm.at[zidx], bvec, sb).wait()

    cp_i2_0.wait()
    flatten(0)
    cp_g = start_gather(0)

    for k in range(NCHUNK):
        s = k % 2
        if k + 1 < NCHUNK:
            cp_i2.wait()
            flatten(k + 1)
        cp_g.wait()
        if k + 1 < NCHUNK:
            cp_g = start_gather(k + 1)
        if k + 2 < NCHUNK:
            cp_i2 = start_i2(k + 2)
        cp_v.wait()
        if k + 1 < NCHUNK:
            cp_v = start_val(k + 1)

        g_v, v_v = g_b[s], v_b[s]

        def grp_body(g, carry2, g_v=g_v, v_v=v_v, k=k):
            row = g * 16 + lane
            row_l = row * L
            acc = bvec[...]
            for slot in range(L):
                col = jnp.full((16,), slot, jnp.int32)
                gv = plsc.load_gather(g_v, [row_l + slot])
                vv = plsc.load_gather(v_v, [row, col])
                acc = acc + gv * vv
            o_v[pl.ds(k * C + g * 16, 16)] = acc
            return carry2

        lax.fori_loop(0, G, grp_body, 0)

    pltpu.sync_copy(o_v, out_hbm.at[pl.ds(wid * RW, RW)])


def kernel(indices, values, emb_table, bias):
    idx = indices.astype(jnp.int32)
    tab = emb_table.reshape(-1)
    return _wide_sc(idx, values, tab, bias)


# speedup vs baseline: 1.4811x; 1.2041x over previous
import functools

import jax
import jax.numpy as jnp
from jax import lax
from jax.experimental import pallas as pl
from jax.experimental.pallas import tpu as pltpu
from jax.experimental.pallas import tpu_sc as plsc

B = 16384
L = 100
NC = 2
NS = 16
NW = NC * NS
RW = B // NW
C = 128
NCHUNK = RW // C
G = C // 16
CL = C * L

_mesh = plsc.VectorSubcoreMesh(core_axis_name="c", subcore_axis_name="s")


@functools.partial(
    pl.kernel,
    mesh=_mesh,
    out_type=jax.ShapeDtypeStruct((B,), jnp.float32),
    scratch_types=[
        pltpu.VMEM((L, C), jnp.int32),
        pltpu.VMEM((L, C), jnp.int32),
        pltpu.VMEM((CL,), jnp.int32),
        pltpu.VMEM((CL,), jnp.int32),
        pltpu.VMEM((CL,), jnp.float32),
        pltpu.VMEM((CL,), jnp.float32),
        pltpu.VMEM((L, C), jnp.float32),
        pltpu.VMEM((L, C), jnp.float32),
        pltpu.VMEM((RW,), jnp.float32),
        pltpu.VMEM((16,), jnp.int32),
        pltpu.VMEM((16,), jnp.float32),
        pltpu.SemaphoreType.DMA,
        pltpu.SemaphoreType.DMA,
        pltpu.SemaphoreType.DMA,
        pltpu.SemaphoreType.DMA,
        pltpu.SemaphoreType.DMA,
        pltpu.SemaphoreType.DMA,
        pltpu.SemaphoreType.DMA,
    ],
    compiler_params=pltpu.CompilerParams(needs_layout_passes=False),
)
def _wide_sc(idx_hbm, val_hbm, tab_hbm, bias_hbm, out_hbm,
             i2_0, i2_1, idx0, idx1, g0, g1, v0, v1, o_v, zidx, bvec,
             si0, si1, sg0, sg1, sv0, sv1, sb):
    cid = lax.axis_index("c")
    sid = lax.axis_index("s")
    wid = sid * NC + cid
    lane = lax.iota(jnp.int32, 16)
    i2_b = (i2_0, i2_1)
    idx_b = (idx0, idx1)
    g_b = (g0, g1)
    v_b = (v0, v1)
    si = (si0, si1)
    sg = (sg0, sg1)
    sv = (sv0, sv1)

    def start_i2(k):
        s = k % 2
        return pltpu.async_copy(
            idx_hbm.at[:, pl.ds(wid * RW + k * C, C)], i2_b[s], si[s])

    def start_val(k):
        s = k % 2
        return pltpu.async_copy(
            val_hbm.at[:, pl.ds(wid * RW + k * C, C)], v_b[s], sv[s])

    def start_gather(k):
        s = k % 2
        return pltpu.async_copy(tab_hbm.at[idx_b[s]], g_b[s], sg[s])

    def flatten(k):
        s = k % 2
        i2, i1 = i2_b[s], idx_b[s]

        def slot_body(l, carry):
            for cb in range(C // 16):
                i1[pl.ds(l * C + cb * 16, 16)] = i2[l, pl.ds(cb * 16, 16)]
            return carry

        lax.fori_loop(0, L, slot_body, 0)

    cp_i2_0 = start_i2(0)
    cp_v = start_val(0)
    cp_i2 = start_i2(1)

    zidx[...] = lane * 0
    pltpu.async_copy(bias_hbm.at[zidx], bvec, sb).wait()

    cp_i2_0.wait()
    flatten(0)
    cp_g = start_gather(0)

    for k in range(NCHUNK):
        s = k % 2
        if k + 1 < NCHUNK:
            cp_i2.wait()
            flatten(k + 1)
        cp_g.wait()
        if k + 1 < NCHUNK:
            cp_g = start_gather(k + 1)
        if k + 2 < NCHUNK:
            cp_i2 = start_i2(k + 2)
        cp_v.wait()
        if k + 1 < NCHUNK:
            cp_v = start_val(k + 1)

        g_v, v_v = g_b[s], v_b[s]

        def grp_body(g, carry2, g_v=g_v, v_v=v_v, k=k):
            acc = bvec[...]
            for slot in range(L):
                gv = g_v[pl.ds(slot * C + g * 16, 16)]
                vv = v_v[slot, pl.ds(g * 16, 16)]
                acc = acc + gv * vv
            o_v[pl.ds(k * C + g * 16, 16)] = acc
            return carry2

        lax.fori_loop(0, G, grp_body, 0)

    pltpu.sync_copy(o_v, out_hbm.at[pl.ds(wid * RW, RW)])


def kernel(indices, values, emb_table, bias):
    idx_t = indices.astype(jnp.int32).T
    val_t = values.T
    tab = emb_table.T.reshape(-1)
    return _wide_sc(idx_t, val_t, tab, bias)


# speedup vs baseline: 1.4847x; 1.0025x over previous
import functools

import jax
import jax.numpy as jnp
from jax import lax
from jax.experimental import pallas as pl
from jax.experimental.pallas import tpu as pltpu
from jax.experimental.pallas import tpu_sc as plsc

B = 16384
L = 100
NC = 2
NS = 16
NW = NC * NS
RW = B // NW
C = 128
NCHUNK = RW // C
G = C // 16
CL = C * L
CH = CL // 2

_mesh = plsc.VectorSubcoreMesh(core_axis_name="c", subcore_axis_name="s")


@functools.partial(
    pl.kernel,
    mesh=_mesh,
    out_type=jax.ShapeDtypeStruct((B,), jnp.float32),
    scratch_types=[
        pltpu.VMEM((L, C), jnp.int32),
        pltpu.VMEM((L, C), jnp.int32),
        pltpu.VMEM((CL,), jnp.int32),
        pltpu.VMEM((CL,), jnp.int32),
        pltpu.VMEM((CL,), jnp.float32),
        pltpu.VMEM((CL,), jnp.float32),
        pltpu.VMEM((L, C), jnp.float32),
        pltpu.VMEM((L, C), jnp.float32),
        pltpu.VMEM((RW,), jnp.float32),
        pltpu.VMEM((16,), jnp.int32),
        pltpu.VMEM((16,), jnp.float32),
        pltpu.SemaphoreType.DMA,
        pltpu.SemaphoreType.DMA,
        pltpu.SemaphoreType.DMA,
        pltpu.SemaphoreType.DMA,
        pltpu.SemaphoreType.DMA,
        pltpu.SemaphoreType.DMA,
        pltpu.SemaphoreType.DMA,
        pltpu.SemaphoreType.DMA,
        pltpu.SemaphoreType.DMA,
    ],
    compiler_params=pltpu.CompilerParams(needs_layout_passes=False),
)
def _wide_sc(idx_hbm, val_hbm, tab_hbm, bias_hbm, out_hbm,
             i2_0, i2_1, idx0, idx1, g0, g1, v0, v1, o_v, zidx, bvec,
             si0, si1, sg0, sg1, sga0, sga1, sv0, sv1, sb):
    cid = lax.axis_index("c")
    sid = lax.axis_index("s")
    wid = sid * NC + cid
    lane = lax.iota(jnp.int32, 16)
    i2_b = (i2_0, i2_1)
    idx_b = (idx0, idx1)
    g_b = (g0, g1)
    v_b = (v0, v1)
    si = (si0, si1)
    sg = (sg0, sg1)
    sga = (sga0, sga1)
    sv = (sv0, sv1)

    def start_i2(k):
        s = k % 2
        return pltpu.async_copy(
            idx_hbm.at[:, pl.ds(wid * RW + k * C, C)], i2_b[s], si[s])

    def start_val(k):
        s = k % 2
        return pltpu.async_copy(
            val_hbm.at[:, pl.ds(wid * RW + k * C, C)], v_b[s], sv[s])

    def start_gather(k):
        s = k % 2
        ca = pltpu.async_copy(
            tab_hbm.at[idx_b[s].at[pl.ds(0, CH)]], g_b[s].at[pl.ds(0, CH)],
            sg[s])
        cb = pltpu.async_copy(
            tab_hbm.at[idx_b[s].at[pl.ds(CH, CH)]], g_b[s].at[pl.ds(CH, CH)],
            sga[s])
        return (ca, cb)

    def flatten(k):
        s = k % 2
        i2, i1 = i2_b[s], idx_b[s]

        def slot_body(l, carry):
            for cb in range(C // 16):
                i1[pl.ds(l * C + cb * 16, 16)] = i2[l, pl.ds(cb * 16, 16)]
            return carry

        lax.fori_loop(0, L, slot_body, 0)

    cp_i2_0 = start_i2(0)
    cp_v = start_val(0)
    cp_i2 = start_i2(1)

    zidx[...] = lane * 0
    pltpu.async_copy(bias_hbm.at[zidx], bvec, sb).wait()

    cp_i2_0.wait()
    flatten(0)
    cp_g = start_gather(0)

    for k in range(NCHUNK):
        s = k % 2
        if k + 1 < NCHUNK:
            cp_i2.wait()
            flatten(k + 1)
        cp_g[0].wait()
        cp_g[1].wait()
        if k + 1 < NCHUNK:
            cp_g = start_gather(k + 1)
        if k + 2 < NCHUNK:
            cp_i2 = start_i2(k + 2)
        cp_v.wait()
        if k + 1 < NCHUNK:
            cp_v = start_val(k + 1)

        g_v, v_v = g_b[s], v_b[s]

        def grp_body(g, carry2, g_v=g_v, v_v=v_v, k=k):
            acc = bvec[...]
            for slot in range(L):
                gv = g_v[pl.ds(slot * C + g * 16, 16)]
                vv = v_v[slot, pl.ds(g * 16, 16)]
                acc = acc + gv * vv
            o_v[pl.ds(k * C + g * 16, 16)] = acc
            return carry2

        lax.fori_loop(0, G, grp_body, 0)

    pltpu.sync_copy(o_v, out_hbm.at[pl.ds(wid * RW, RW)])


def kernel(indices, values, emb_table, bias):
    idx_t = indices.astype(jnp.int32).T
    val_t = values.T
    tab = emb_table.reshape(-1)
    return _wide_sc(idx_t, val_t, tab, bias)
